# baseline (speedup 1.0000x reference)
import jax
import jax.numpy as jnp
from jax.experimental import pallas as pl
from jax.experimental.pallas import tpu as pltpu

_B, _P, _Q, _D = 8, 2048, 2048, 3

_BETA = 1.0
_GAMMA_EFF = 1.0


def _argmin_lanes(d):
    rows, cols = d.shape
    w = cols // 2
    mask = d[:, w:] < d[:, :w]
    v = jnp.where(mask, d[:, w:], d[:, :w])
    base = jax.lax.broadcasted_iota(jnp.int32, (rows, w), 1)
    idx = jnp.where(mask, base + w, base)
    w //= 2
    while w >= 128:
        mask = v[:, w:] < v[:, :w]
        v = jnp.where(mask, v[:, w:], v[:, :w])
        idx = jnp.where(mask, idx[:, w:], idx[:, :w])
        w //= 2
    m = jnp.min(v, axis=1, keepdims=True)
    i = jnp.min(jnp.where(v == m, idx, cols), axis=1, keepdims=True)
    return m, i


def _argmin_sublanes(d):
    rows, cols = d.shape
    h = rows // 2
    mask = d[h:, :] < d[:h, :]
    v = jnp.where(mask, d[h:, :], d[:h, :])
    base = jax.lax.broadcasted_iota(jnp.int32, (h, cols), 0)
    idx = jnp.where(mask, base + h, base)
    h //= 2
    while h >= 8:
        mask = v[h:, :] < v[:h, :]
        v = jnp.where(mask, v[h:, :], v[:h, :])
        idx = jnp.where(mask, idx[h:, :], idx[:h, :])
        h //= 2
    m = jnp.min(v, axis=0, keepdims=True)
    i = jnp.min(jnp.where(v == m, idx, rows), axis=0, keepdims=True)
    return m, i


def _chamfer_body(x1_ref, x2_ref, loss_ref, idx12_ref, idx21_ref):
    b = pl.program_id(0)
    x1 = x1_ref[0]
    x2 = x2_ref[0]
    s1 = jnp.sum(x1 * x1, axis=1, keepdims=True)
    s2 = jnp.sum(x2 * x2, axis=1, keepdims=True)
    s1r = s1.reshape(1, _P)
    s2r = s2.reshape(1, _Q)
    x1d = x1 + x1
    x2d = x2 + x2

    inner2 = jax.lax.dot_general(
        x1d, x2, (((1,), (1,)), ((), ())),
        precision=jax.lax.Precision.DEFAULT,
        preferred_element_type=jnp.float32)
    d = (s1 + s2r) - inner2
    m21, i21 = _argmin_sublanes(d)
    idx21_ref[0] = i21

    inner2t = jax.lax.dot_general(
        x2d, x1, (((1,), (1,)), ((), ())),
        precision=jax.lax.Precision.DEFAULT,
        preferred_element_type=jnp.float32)
    dt = (s2 + s1r) - inner2t
    min12, idx12 = _argmin_sublanes(dt)
    idx12_ref[0] = idx12
    part = (jnp.sum(min12) / _P
            + _BETA * jnp.max(min12)
            + _GAMMA_EFF * jnp.sum(m21) / _Q).reshape(1, 1)

    @pl.when(b == 0)
    def _():
        loss_ref[...] = part

    @pl.when(b > 0)
    def _():
        loss_ref[...] = loss_ref[...] + part

    @pl.when(b == _B - 1)
    def _():
        loss_ref[...] = loss_ref[...] * (1.0 / _B)


def kernel(xyz1, xyz2):
    loss2d, idx12, idx21 = pl.pallas_call(
        _chamfer_body,
        grid=(_B,),
        in_specs=[
            pl.BlockSpec((1, _P, _D), lambda b: (b, 0, 0)),
            pl.BlockSpec((1, _Q, _D), lambda b: (b, 0, 0)),
        ],
        out_specs=[
            pl.BlockSpec((1, 1), lambda b: (0, 0)),
            pl.BlockSpec((1, 1, _P), lambda b: (b, 0, 0)),
            pl.BlockSpec((1, 1, _Q), lambda b: (b, 0, 0)),
        ],
        out_shape=[
            jax.ShapeDtypeStruct((1, 1), jnp.float32),
            jax.ShapeDtypeStruct((_B, 1, _P), jnp.int32),
            jax.ShapeDtypeStruct((_B, 1, _Q), jnp.int32),
        ],
        compiler_params=pltpu.CompilerParams(
            dimension_semantics=("arbitrary",)),
    )(xyz1, xyz2)
    return loss2d[0, 0], idx12.reshape(_B, _P), idx21.reshape(_B, _Q)


# speedup vs baseline: 1.0237x; 1.0237x over previous
import jax
import jax.numpy as jnp
from jax.experimental import pallas as pl
from jax.experimental.pallas import tpu as pltpu

_B, _P, _Q, _D = 8, 2048, 2048, 3

_BETA = 1.0
_GAMMA_EFF = 1.0


def _argmin_lanes(d):
    rows, cols = d.shape
    w = cols // 2
    mask = d[:, w:] < d[:, :w]
    v = jnp.where(mask, d[:, w:], d[:, :w])
    base = jax.lax.broadcasted_iota(jnp.int32, (rows, w), 1)
    idx = jnp.where(mask, base + w, base)
    w //= 2
    while w >= 128:
        mask = v[:, w:] < v[:, :w]
        v = jnp.where(mask, v[:, w:], v[:, :w])
        idx = jnp.where(mask, idx[:, w:], idx[:, :w])
        w //= 2
    m = jnp.min(v, axis=1, keepdims=True)
    i = jnp.min(jnp.where(v == m, idx, cols), axis=1, keepdims=True)
    return m, i


def _argmin_sublanes(d):
    rows, cols = d.shape
    h = rows // 2
    mask = d[h:, :] < d[:h, :]
    v = jnp.where(mask, d[h:, :], d[:h, :])
    base = jax.lax.broadcasted_iota(jnp.int32, (h, cols), 0)
    idx = jnp.where(mask, base + h, base)
    h //= 2
    while h >= 8:
        mask = v[h:, :] < v[:h, :]
        v = jnp.where(mask, v[h:, :], v[:h, :])
        idx = jnp.where(mask, idx[h:, :], idx[:h, :])
        h //= 2
    m = jnp.min(v, axis=0, keepdims=True)
    i = jnp.min(jnp.where(v == m, idx, rows), axis=0, keepdims=True)
    return m, i


def _chamfer_body(x1_ref, x2_ref, loss_ref, idx12_ref, idx21_ref):
    b = pl.program_id(0)
    x1 = x1_ref[0]
    x2 = x2_ref[0]
    s1 = jnp.sum(x1 * x1, axis=1, keepdims=True)
    s2 = jnp.sum(x2 * x2, axis=1, keepdims=True)
    s1r = s1.reshape(1, _P)
    s2r = s2.reshape(1, _Q)
    x1d = x1 + x1
    x2d = x2 + x2

    inner2 = jax.lax.dot_general(
        x1d, x2, (((1,), (1,)), ((), ())),
        precision=jax.lax.Precision.DEFAULT,
        preferred_element_type=jnp.float32)
    d = (s1 + s2r) - inner2
    m21, i21 = _argmin_sublanes(d)
    idx21_ref[pl.dslice(b, 1), :] = i21

    inner2t = jax.lax.dot_general(
        x2d, x1, (((1,), (1,)), ((), ())),
        precision=jax.lax.Precision.DEFAULT,
        preferred_element_type=jnp.float32)
    dt = (s2 + s1r) - inner2t
    min12, idx12 = _argmin_sublanes(dt)
    idx12_ref[pl.dslice(b, 1), :] = idx12
    part = (jnp.sum(min12) / _P
            + _BETA * jnp.max(min12)
            + _GAMMA_EFF * jnp.sum(m21) / _Q).reshape(1, 1)

    @pl.when(b == 0)
    def _():
        loss_ref[...] = part

    @pl.when(b > 0)
    def _():
        loss_ref[...] = loss_ref[...] + part

    @pl.when(b == _B - 1)
    def _():
        loss_ref[...] = loss_ref[...] * (1.0 / _B)


def kernel(xyz1, xyz2):
    loss2d, idx12, idx21 = pl.pallas_call(
        _chamfer_body,
        grid=(_B,),
        in_specs=[
            pl.BlockSpec((1, _P, _D), lambda b: (b, 0, 0)),
            pl.BlockSpec((1, _Q, _D), lambda b: (b, 0, 0)),
        ],
        out_specs=[
            pl.BlockSpec((1, 1), lambda b: (0, 0)),
            pl.BlockSpec((_B, _P), lambda b: (0, 0)),
            pl.BlockSpec((_B, _Q), lambda b: (0, 0)),
        ],
        out_shape=[
            jax.ShapeDtypeStruct((1, 1), jnp.float32),
            jax.ShapeDtypeStruct((_B, _P), jnp.int32),
            jax.ShapeDtypeStruct((_B, _Q), jnp.int32),
        ],
        compiler_params=pltpu.CompilerParams(
            dimension_semantics=("arbitrary",)),
    )(xyz1, xyz2)
    return loss2d[0, 0], idx12, idx21


# speedup vs baseline: 1.0244x; 1.0007x over previous
import jax
import jax.numpy as jnp
from jax.experimental import pallas as pl
from jax.experimental.pallas import tpu as pltpu

_B, _P, _Q, _D = 8, 2048, 2048, 3

_BETA = 1.0
_GAMMA_EFF = 1.0


def _argmin_sublanes(d):
    rows, cols = d.shape
    h = rows // 2
    mask = d[h:, :] < d[:h, :]
    v = jnp.where(mask, d[h:, :], d[:h, :])
    base = jax.lax.broadcasted_iota(jnp.int32, (h, cols), 0)
    idx = jnp.where(mask, base + h, base)
    h //= 2
    while h >= 8:
        mask = v[h:, :] < v[:h, :]
        v = jnp.where(mask, v[h:, :], v[:h, :])
        idx = jnp.where(mask, idx[h:, :], idx[:h, :])
        h //= 2
    m = jnp.min(v, axis=0, keepdims=True)
    i = jnp.min(jnp.where(v == m, idx, rows), axis=0, keepdims=True)
    return m, i


_W = 256


def _dir_min_strips(xo_d, s_o, x_ref, s_r, out_ref, b):
    n_strips = s_r.shape[1] // _W
    mins = []
    for j in range(n_strips):
        xs = x_ref[0, pl.dslice(j * _W, _W), :]
        inner = jax.lax.dot_general(
            xo_d, xs, (((1,), (1,)), ((), ())),
            precision=jax.lax.Precision.DEFAULT,
            preferred_element_type=jnp.float32)
        dstr = (s_o + s_r[:, j * _W:(j + 1) * _W]) - inner
        m, i = _argmin_sublanes(dstr)
        out_ref[pl.dslice(b, 1), pl.dslice(j * _W, _W)] = i
        mins.append(m)
    return jnp.concatenate(mins, axis=1)


def _chamfer_body(x1_ref, x2_ref, loss_ref, idx12_ref, idx21_ref):
    b = pl.program_id(0)
    x1 = x1_ref[0]
    x2 = x2_ref[0]
    s1 = jnp.sum(x1 * x1, axis=1, keepdims=True)
    s2 = jnp.sum(x2 * x2, axis=1, keepdims=True)
    s1r = s1.reshape(1, _P)
    s2r = s2.reshape(1, _Q)
    x1d = x1 + x1
    x2d = x2 + x2

    m21 = _dir_min_strips(x1d, s1, x2_ref, s2r, idx21_ref, b)

    min12 = _dir_min_strips(x2d, s2, x1_ref, s1r, idx12_ref, b)

    part = (jnp.sum(min12) / _P
            + _BETA * jnp.max(min12)
            + _GAMMA_EFF * jnp.sum(m21) / _Q).reshape(1, 1)

    @pl.when(b == 0)
    def _():
        loss_ref[...] = part

    @pl.when(b > 0)
    def _():
        loss_ref[...] = loss_ref[...] + part

    @pl.when(b == _B - 1)
    def _():
        loss_ref[...] = loss_ref[...] * (1.0 / _B)


def kernel(xyz1, xyz2):
    loss2d, idx12, idx21 = pl.pallas_call(
        _chamfer_body,
        grid=(_B,),
        in_specs=[
            pl.BlockSpec((1, _P, _D), lambda b: (b, 0, 0)),
            pl.BlockSpec((1, _Q, _D), lambda b: (b, 0, 0)),
        ],
        out_specs=[
            pl.BlockSpec((1, 1), lambda b: (0, 0)),
            pl.BlockSpec((_B, _P), lambda b: (0, 0)),
            pl.BlockSpec((_B, _Q), lambda b: (0, 0)),
        ],
        out_shape=[
            jax.ShapeDtypeStruct((1, 1), jnp.float32),
            jax.ShapeDtypeStruct((_B, _P), jnp.int32),
            jax.ShapeDtypeStruct((_B, _Q), jnp.int32),
        ],
        compiler_params=pltpu.CompilerParams(
            dimension_semantics=("arbitrary",)),
    )(xyz1, xyz2)
    return loss2d[0, 0], idx12, idx21


# speedup vs baseline: 1.0256x; 1.0012x over previous
import jax
import jax.numpy as jnp
from jax.experimental import pallas as pl
from jax.experimental.pallas import tpu as pltpu

_B, _P, _Q, _D = 8, 2048, 2048, 3

_BETA = 1.0
_GAMMA_EFF = 1.0


def _argmin_sublanes(d):
    rows, cols = d.shape
    h = rows // 2
    mask = d[h:, :] < d[:h, :]
    v = jnp.where(mask, d[h:, :], d[:h, :])
    base = jax.lax.broadcasted_iota(jnp.int32, (h, cols), 0)
    idx = jnp.where(mask, base + h, base)
    h //= 2
    while h >= 8:
        mask = v[h:, :] < v[:h, :]
        v = jnp.where(mask, v[h:, :], v[:h, :])
        idx = jnp.where(mask, idx[h:, :], idx[:h, :])
        h //= 2
    m = jnp.min(v, axis=0, keepdims=True)
    i = jnp.min(jnp.where(v == m, idx, rows), axis=0, keepdims=True)
    return m, i


_W = 512


def _dir_min_strips(xo_d, s_o, x_ref, s_r, out_ref, b):
    n_strips = s_r.shape[1] // _W
    mins = []
    for j in range(n_strips):
        xs = x_ref[0, pl.dslice(j * _W, _W), :]
        inner = jax.lax.dot_general(
            xo_d, xs, (((1,), (1,)), ((), ())),
            precision=jax.lax.Precision.DEFAULT,
            preferred_element_type=jnp.float32)
        dstr = (s_o + s_r[:, j * _W:(j + 1) * _W]) - inner
        m, i = _argmin_sublanes(dstr)
        out_ref[pl.dslice(b, 1), pl.dslice(j * _W, _W)] = i
        mins.append(m)
    return jnp.concatenate(mins, axis=1)


def _chamfer_body(x1_ref, x2_ref, loss_ref, idx12_ref, idx21_ref):
    b = pl.program_id(0)
    x1 = x1_ref[0]
    x2 = x2_ref[0]
    s1 = jnp.sum(x1 * x1, axis=1, keepdims=True)
    s2 = jnp.sum(x2 * x2, axis=1, keepdims=True)
    s1r = s1.reshape(1, _P)
    s2r = s2.reshape(1, _Q)
    x1d = x1 + x1
    x2d = x2 + x2

    m21 = _dir_min_strips(x1d, s1, x2_ref, s2r, idx21_ref, b)

    min12 = _dir_min_strips(x2d, s2, x1_ref, s1r, idx12_ref, b)

    part = (jnp.sum(min12) / _P
            + _BETA * jnp.max(min12)
            + _GAMMA_EFF * jnp.sum(m21) / _Q).reshape(1, 1)

    @pl.when(b == 0)
    def _():
        loss_ref[...] = part

    @pl.when(b > 0)
    def _():
        loss_ref[...] = loss_ref[...] + part

    @pl.when(b == _B - 1)
    def _():
        loss_ref[...] = loss_ref[...] * (1.0 / _B)


def kernel(xyz1, xyz2):
    loss2d, idx12, idx21 = pl.pallas_call(
        _chamfer_body,
        grid=(_B,),
        in_specs=[
            pl.BlockSpec((1, _P, _D), lambda b: (b, 0, 0)),
            pl.BlockSpec((1, _Q, _D), lambda b: (b, 0, 0)),
        ],
        out_specs=[
            pl.BlockSpec((1, 1), lambda b: (0, 0)),
            pl.BlockSpec((_B, _P), lambda b: (0, 0)),
            pl.BlockSpec((_B, _Q), lambda b: (0, 0)),
        ],
        out_shape=[
            jax.ShapeDtypeStruct((1, 1), jnp.float32),
            jax.ShapeDtypeStruct((_B, _P), jnp.int32),
            jax.ShapeDtypeStruct((_B, _Q), jnp.int32),
        ],
        compiler_params=pltpu.CompilerParams(
            dimension_semantics=("arbitrary",)),
    )(xyz1, xyz2)
    return loss2d[0, 0], idx12, idx21


# speedup vs baseline: 1.0395x; 1.0136x over previous
import jax
import jax.numpy as jnp
from jax.experimental import pallas as pl
from jax.experimental.pallas import tpu as pltpu

_B, _P, _Q, _D = 8, 2048, 2048, 3

_BETA = 1.0
_GAMMA_EFF = 1.0


def _argmin_sublanes(d):
    rows, cols = d.shape
    h = rows // 2
    mask = d[h:, :] < d[:h, :]
    v = jnp.where(mask, d[h:, :], d[:h, :])
    base = jax.lax.broadcasted_iota(jnp.int32, (h, cols), 0)
    idx = jnp.where(mask, base + h, base)
    h //= 2
    while h >= 8:
        mask = v[h:, :] < v[:h, :]
        v = jnp.where(mask, v[h:, :], v[:h, :])
        idx = jnp.where(mask, idx[h:, :], idx[:h, :])
        h //= 2
    m = jnp.min(v, axis=0, keepdims=True)
    i = jnp.min(jnp.where(v == m, idx, rows), axis=0, keepdims=True)
    return m, i


_W = 512


def _dir_min_strips(xo_d, s_o, x_ref, s_r, out_ref, b, b2):
    n_strips = s_r.shape[1] // _W
    mins = []
    for j in range(n_strips):
        xs = x_ref[b2, pl.dslice(j * _W, _W), :]
        inner = jax.lax.dot_general(
            xo_d, xs, (((1,), (1,)), ((), ())),
            precision=jax.lax.Precision.DEFAULT,
            preferred_element_type=jnp.float32)
        dstr = (s_o + s_r[:, j * _W:(j + 1) * _W]) - inner
        m, i = _argmin_sublanes(dstr)
        out_ref[pl.dslice(b, 1), pl.dslice(j * _W, _W)] = i
        mins.append(m)
    return jnp.concatenate(mins, axis=1)


_BB = 2


def _chamfer_body(x1_ref, x2_ref, loss_ref, idx12_ref, idx21_ref):
    g = pl.program_id(0)
    part = None
    for b2 in range(_BB):
        row = g * _BB + b2
        x1 = x1_ref[b2]
        x2 = x2_ref[b2]
        s1 = jnp.sum(x1 * x1, axis=1, keepdims=True)
        s2 = jnp.sum(x2 * x2, axis=1, keepdims=True)
        s1r = s1.reshape(1, _P)
        s2r = s2.reshape(1, _Q)
        x1d = x1 + x1
        x2d = x2 + x2

        m21 = _dir_min_strips(x1d, s1, x2_ref, s2r, idx21_ref, row, b2)

        min12 = _dir_min_strips(x2d, s2, x1_ref, s1r, idx12_ref, row, b2)

        p = (jnp.sum(min12) / _P
             + _BETA * jnp.max(min12)
             + _GAMMA_EFF * jnp.sum(m21) / _Q).reshape(1, 1)
        part = p if part is None else part + p

    @pl.when(g == 0)
    def _():
        loss_ref[...] = part

    @pl.when(g > 0)
    def _():
        loss_ref[...] = loss_ref[...] + part

    @pl.when(g == (_B // _BB) - 1)
    def _():
        loss_ref[...] = loss_ref[...] * (1.0 / _B)


def kernel(xyz1, xyz2):
    loss2d, idx12, idx21 = pl.pallas_call(
        _chamfer_body,
        grid=(_B // _BB,),
        in_specs=[
            pl.BlockSpec((_BB, _P, _D), lambda b: (b, 0, 0)),
            pl.BlockSpec((_BB, _Q, _D), lambda b: (b, 0, 0)),
        ],
        out_specs=[
            pl.BlockSpec((1, 1), lambda b: (0, 0)),
            pl.BlockSpec((_B, _P), lambda b: (0, 0)),
            pl.BlockSpec((_B, _Q), lambda b: (0, 0)),
        ],
        out_shape=[
            jax.ShapeDtypeStruct((1, 1), jnp.float32),
            jax.ShapeDtypeStruct((_B, _P), jnp.int32),
            jax.ShapeDtypeStruct((_B, _Q), jnp.int32),
        ],
        compiler_params=pltpu.CompilerParams(
            dimension_semantics=("arbitrary",)),
    )(xyz1, xyz2)
    return loss2d[0, 0], idx12, idx21
